# speedup vs baseline: 9.9676x; 9.9676x over previous
import functools

import jax
import jax.numpy as jnp
from jax.experimental import pallas as pl
from jax.experimental.pallas import tpu as pltpu

_K = 10
_THRESH = 0.1


def _composite_kernel(s_ref, alpha_ref, color_ref, canvas_ref, den_ref,
                      T_ref, C_ref, D_ref, cnt_ref, *, n_chunk, num_chunks):
    j = pl.program_id(1)

    @pl.when(j == 0)
    def _init():
        T_ref[...] = jnp.ones_like(T_ref)
        C_ref[...] = jnp.zeros_like(C_ref)
        D_ref[...] = jnp.zeros_like(D_ref)
        cnt_ref[...] = jnp.zeros_like(cnt_ref)

    T = T_ref[...]
    cnt = cnt_ref[...]
    C0, C1, C2 = C_ref[0], C_ref[1], C_ref[2]
    D = D_ref[...]

    def body(i, carry):
        T, cnt, C0, C1, C2, D = carry
        idx = n_chunk - 1 - i
        a = alpha_ref[0, idx, 0]
        take = (a > _THRESH) & (cnt < _K)
        w = jnp.where(take, T * a, 0.0)
        C0 = C0 + w * color_ref[0, idx, 0]
        C1 = C1 + w * color_ref[0, idx, 1]
        C2 = C2 + w * color_ref[0, idx, 2]
        D = D + w * s_ref[0, idx, 0, 0]
        T = jnp.where(take, T * (1.0 - a), T)
        cnt = cnt + take.astype(jnp.int32)
        return T, cnt, C0, C1, C2, D

    T, cnt, C0, C1, C2, D = jax.lax.fori_loop(
        0, n_chunk, body, (T, cnt, C0, C1, C2, D))

    T_ref[...] = T
    cnt_ref[...] = cnt
    C_ref[0] = C0
    C_ref[1] = C1
    C_ref[2] = C2
    D_ref[...] = D

    @pl.when(j == num_chunks - 1)
    def _finish():
        canvas_ref[0, 0] = C0 + T
        canvas_ref[0, 1] = C1 + T
        canvas_ref[0, 2] = C2 + T
        den_ref[0, 0] = D + T


@jax.jit
def kernel(color_stroke, alpha, params):
    b, n = color_stroke.shape[0], color_stroke.shape[1]
    w = color_stroke.shape[-1]
    n_chunk = 8
    num_chunks = n // n_chunk

    s = (params[:, :, 2] * params[:, :, 3]).reshape(b, n, 1, 1)

    kfn = functools.partial(_composite_kernel, n_chunk=n_chunk,
                            num_chunks=num_chunks)
    grid = (b, num_chunks)

    canvas, den = pl.pallas_call(
        kfn,
        grid=grid,
        in_specs=[
            pl.BlockSpec((1, n_chunk, 1, 1),
                         lambda bi, j: (bi, num_chunks - 1 - j, 0, 0)),
            pl.BlockSpec((1, n_chunk, 1, w, w),
                         lambda bi, j: (bi, num_chunks - 1 - j, 0, 0, 0)),
            pl.BlockSpec((1, n_chunk, 3, w, w),
                         lambda bi, j: (bi, num_chunks - 1 - j, 0, 0, 0)),
        ],
        out_specs=[
            pl.BlockSpec((1, 3, w, w), lambda bi, j: (bi, 0, 0, 0)),
            pl.BlockSpec((1, 1, w, w), lambda bi, j: (bi, 0, 0, 0)),
        ],
        out_shape=[
            jax.ShapeDtypeStruct((b, 3, w, w), jnp.float32),
            jax.ShapeDtypeStruct((b, 1, w, w), jnp.float32),
        ],
        scratch_shapes=[
            pltpu.VMEM((w, w), jnp.float32),
            pltpu.VMEM((3, w, w), jnp.float32),
            pltpu.VMEM((w, w), jnp.float32),
            pltpu.VMEM((w, w), jnp.int32),
        ],
        compiler_params=pltpu.CompilerParams(
            dimension_semantics=("arbitrary", "arbitrary"),
        ),
    )(s, alpha, color_stroke)

    return (canvas, den)


# speedup vs baseline: 59.0512x; 5.9243x over previous
import functools

import jax
import jax.numpy as jnp
from jax.experimental import pallas as pl
from jax.experimental.pallas import tpu as pltpu

_K = 10
_THRESH = 0.1


def _composite_kernel(s_ref, alpha_hbm, color_hbm, canvas_ref, den_ref,
                      abuf, cbuf, T_ref, C_ref, D_ref, cnt_ref, sem,
                      *, ch, num_chunks):
    b = pl.program_id(0)

    T_ref[...] = jnp.ones_like(T_ref)
    C_ref[...] = jnp.zeros_like(C_ref)
    D_ref[...] = jnp.zeros_like(D_ref)
    cnt_ref[...] = jnp.zeros_like(cnt_ref)

    def a_copy(j, slot):
        start = (num_chunks - 1 - j) * ch
        return pltpu.make_async_copy(
            alpha_hbm.at[b, pl.ds(start, ch)], abuf.at[slot], sem.at[slot, 0])

    def c_copy(j, slot):
        start = (num_chunks - 1 - j) * ch
        return pltpu.make_async_copy(
            color_hbm.at[b, pl.ds(start, ch)], cbuf.at[slot], sem.at[slot, 1])

    def start_copies(j, slot):
        a_copy(j, slot).start()
        c_copy(j, slot).start()

    def wait_copies(j, slot):
        a_copy(j, slot).wait()
        c_copy(j, slot).wait()

    start_copies(0, 0)

    def cond(state):
        j, done = state
        return jnp.logical_and(jnp.logical_not(done), j < num_chunks)

    def body(state):
        j, _ = state
        slot = jax.lax.rem(j, 2)

        @pl.when(j + 1 < num_chunks)
        def _prefetch():
            start_copies(j + 1, 1 - slot)

        wait_copies(j, slot)

        base = (num_chunks - 1 - j) * ch
        T = T_ref[...]
        cnt = cnt_ref[...]
        C0, C1, C2 = C_ref[0], C_ref[1], C_ref[2]
        D = D_ref[...]

        def sbody(i, carry):
            T, cnt, C0, C1, C2, D = carry
            idx = ch - 1 - i
            a = abuf[slot, idx, 0]
            take = (a > _THRESH) & (cnt < _K)
            w = jnp.where(take, T * a, 0.0)
            C0 = C0 + w * cbuf[slot, idx, 0]
            C1 = C1 + w * cbuf[slot, idx, 1]
            C2 = C2 + w * cbuf[slot, idx, 2]
            D = D + w * s_ref[0, base + idx, 0, 0]
            T = T - w
            cnt = cnt + take.astype(jnp.int32)
            return T, cnt, C0, C1, C2, D

        T, cnt, C0, C1, C2, D = jax.lax.fori_loop(
            0, ch, sbody, (T, cnt, C0, C1, C2, D))

        T_ref[...] = T
        cnt_ref[...] = cnt
        C_ref[0] = C0
        C_ref[1] = C1
        C_ref[2] = C2
        D_ref[...] = D

        done = jnp.min(cnt) >= _K
        return j + 1, done

    jf, _ = jax.lax.while_loop(cond, body, (jnp.int32(0), jnp.bool_(False)))

    @pl.when(jf < num_chunks)
    def _drain():
        wait_copies(jf, jax.lax.rem(jf, 2))

    @pl.when(jnp.min(cnt_ref[...]) < _K)
    def _tie_fill():
        def a_copy2(j, slot):
            return pltpu.make_async_copy(
                alpha_hbm.at[b, pl.ds(j * ch, ch)], abuf.at[slot],
                sem.at[slot, 0])

        def c_copy2(j, slot):
            return pltpu.make_async_copy(
                color_hbm.at[b, pl.ds(j * ch, ch)], cbuf.at[slot],
                sem.at[slot, 1])

        a_copy2(0, 0).start()
        c_copy2(0, 0).start()

        def cond2(state):
            j, done = state
            return jnp.logical_and(jnp.logical_not(done), j < num_chunks)

        def body2(state):
            j, _ = state
            slot = jax.lax.rem(j, 2)

            @pl.when(j + 1 < num_chunks)
            def _prefetch2():
                a_copy2(j + 1, 1 - slot).start()
                c_copy2(j + 1, 1 - slot).start()

            a_copy2(j, slot).wait()
            c_copy2(j, slot).wait()

            base = j * ch
            T = T_ref[...]
            cnt = cnt_ref[...]
            C0, C1, C2 = C_ref[0], C_ref[1], C_ref[2]
            D = D_ref[...]

            def sbody2(i, carry):
                T, cnt, C0, C1, C2, D = carry
                a = abuf[slot, i, 0]
                take = (a <= _THRESH) & (cnt < _K)
                w = jnp.where(take, T * a, 0.0)
                C0 = C0 + w * cbuf[slot, i, 0]
                C1 = C1 + w * cbuf[slot, i, 1]
                C2 = C2 + w * cbuf[slot, i, 2]
                T = T - w
                cnt = cnt + take.astype(jnp.int32)
                return T, cnt, C0, C1, C2, D

            T, cnt, C0, C1, C2, D = jax.lax.fori_loop(
                0, ch, sbody2, (T, cnt, C0, C1, C2, D))

            T_ref[...] = T
            cnt_ref[...] = cnt
            C_ref[0] = C0
            C_ref[1] = C1
            C_ref[2] = C2
            D_ref[...] = D

            done = jnp.min(cnt) >= _K
            return j + 1, done

        jf2, _ = jax.lax.while_loop(cond2, body2,
                                    (jnp.int32(0), jnp.bool_(False)))

        @pl.when(jf2 < num_chunks)
        def _drain2():
            slot = jax.lax.rem(jf2, 2)
            a_copy2(jf2, slot).wait()
            c_copy2(jf2, slot).wait()

    T = T_ref[...]
    canvas_ref[0, 0] = C_ref[0] + T
    canvas_ref[0, 1] = C_ref[1] + T
    canvas_ref[0, 2] = C_ref[2] + T
    den_ref[0, 0] = D_ref[...] + T


@jax.jit
def kernel(color_stroke, alpha, params):
    b, n = color_stroke.shape[0], color_stroke.shape[1]
    w = color_stroke.shape[-1]
    ch = 8
    num_chunks = n // ch

    s = (params[:, :, 2] * params[:, :, 3]).reshape(b, n, 1, 1)

    kfn = functools.partial(_composite_kernel, ch=ch, num_chunks=num_chunks)

    canvas, den = pl.pallas_call(
        kfn,
        grid=(b,),
        in_specs=[
            pl.BlockSpec((1, n, 1, 1), lambda bi: (bi, 0, 0, 0)),
            pl.BlockSpec(memory_space=pl.ANY),
            pl.BlockSpec(memory_space=pl.ANY),
        ],
        out_specs=[
            pl.BlockSpec((1, 3, w, w), lambda bi: (bi, 0, 0, 0)),
            pl.BlockSpec((1, 1, w, w), lambda bi: (bi, 0, 0, 0)),
        ],
        out_shape=[
            jax.ShapeDtypeStruct((b, 3, w, w), jnp.float32),
            jax.ShapeDtypeStruct((b, 1, w, w), jnp.float32),
        ],
        scratch_shapes=[
            pltpu.VMEM((2, ch, 1, w, w), jnp.float32),
            pltpu.VMEM((2, ch, 3, w, w), jnp.float32),
            pltpu.VMEM((w, w), jnp.float32),
            pltpu.VMEM((3, w, w), jnp.float32),
            pltpu.VMEM((w, w), jnp.float32),
            pltpu.VMEM((w, w), jnp.int32),
            pltpu.SemaphoreType.DMA((2, 2)),
        ],
        compiler_params=pltpu.CompilerParams(
            dimension_semantics=("arbitrary",),
        ),
    )(s, alpha, color_stroke)

    return (canvas, den)
